# baseline (speedup 1.0000x reference)
import jax
import jax.numpy as jnp
from jax import lax
from jax.experimental import pallas as pl
from jax.experimental.pallas import tpu as pltpu
from jax.experimental.pallas import tpu_sc as plsc

N = 10000
D = 128
C = 128
E = 320000
NEG = 0.2

NPAD = 10240
HALF = NPAD // 2
ACC_C = 128
NW = 32
CHUNK = 128
Q_CHUNKS = 168
EPAD = 16 * Q_CHUNKS * CHUNK
ETOT = E + N
NPADE = EPAD - ETOT

NROW = EPAD // CHUNK
SUPER = 24
NSUPER = Q_CHUNKS // SUPER
NPAIR = SUPER // 2


def _proj_body(x_ref, wl_ref, wr_ref, xl_ref, xr_ref):
    xv = x_ref[...]
    xl_ref[...] = jnp.dot(xv, wl_ref[...], preferred_element_type=jnp.float32)
    xr_ref[...] = jnp.dot(xv, wr_ref[...], preferred_element_type=jnp.float32)


def _project(x_pad, W_l, W_r):
    blk = 1280
    return pl.pallas_call(
        _proj_body,
        grid=(NPAD // blk,),
        in_specs=[
            pl.BlockSpec((blk, D), lambda i: (i, 0)),
            pl.BlockSpec((D, C), lambda i: (0, 0)),
            pl.BlockSpec((D, C), lambda i: (0, 0)),
        ],
        out_specs=[
            pl.BlockSpec((blk, C), lambda i: (i, 0)),
            pl.BlockSpec((blk, C), lambda i: (i, 0)),
        ],
        out_shape=[
            jax.ShapeDtypeStruct((NPAD, C), jnp.float32),
            jax.ShapeDtypeStruct((NPAD, C), jnp.float32),
        ],
    )(x_pad, W_l, W_r)


CMAX = SUPER * CHUNK + 3 * CHUNK


def _edge_body(xl_hbm, xr_hbm, src_hbm, dst_hbm, att_hbm, acc_out, den_out,
               src_big, dst_big, csrc, cdst, clocal, sidx2d,
               xj_a, xi_a, xj_b, xi_b, att_v, ptmp, den_l,
               acc_s, semj_a, semi_a, semj_b, semi_b, sems_a, sems_b):
    cid = lax.axis_index("c")
    sid = lax.axis_index("s")
    wid = cid * 16 + sid
    row_lo = cid * HALF
    tile_row0 = sid * Q_CHUNKS
    zero16 = jnp.zeros((16,), jnp.float32)
    iota16 = lax.iota(jnp.int32, 16)
    acc_slices = [(sid * (HALF // 16), 128),
                  (sid * (HALF // 16) + 128, 128),
                  (sid * (HALF // 16) + 256, 64)]

    def _zrow(i, _):
        for k in range(ACC_C // 16):
            xj_a[i, pl.ds(k * 16, 16)] = zero16
        return 0

    lax.fori_loop(0, CHUNK, _zrow, 0)

    def _zden(i, _):
        den_l[pl.ds(i * 16, 16)] = zero16
        return 0

    lax.fori_loop(0, HALF // 16, _zden, 0)
    for lo, nrows in acc_slices:
        pltpu.sync_copy(xj_a.at[pl.ds(0, nrows)], acc_s.at[pl.ds(lo, nrows)])
    pltpu.sync_copy(att_hbm, att_v)
    plsc.subcore_barrier()

    att_regs = [att_v[pl.ds(k * 16, 16)] for k in range(C // 16)]

    def _gather_descs(r, xjv, xiv, semj, semi):
        off = pl.multiple_of(r * CHUNK, CHUNK)
        dj = pltpu.make_async_copy(
            xl_hbm.at[plsc.Indices(csrc.at[pl.ds(off, CHUNK)],
                                   ignored_value=-1)],
            xjv, semj)
        di = pltpu.make_async_copy(
            xr_hbm.at[plsc.Indices(cdst.at[pl.ds(off, CHUNK)],
                                   ignored_value=-1)],
            xiv, semi)
        return dj, di

    def _fire_gather(r, xjv, xiv, semj, semi):
        dj, di = _gather_descs(r, xjv, xiv, semj, semi)
        dj.start()
        di.start()

    def _wait_gather(r, xjv, xiv, semj, semi):
        dj, di = _gather_descs(r, xjv, xiv, semj, semi)
        dj.wait()
        di.wait()

    def _scatter_desc(b, xjv, sems):
        return pltpu.make_async_copy(
            xjv, acc_s.at[plsc.Indices(sidx2d.at[b], ignored_value=-1)],
            sems)

    def _stage_scatter_idx(r, b):
        off = pl.multiple_of(r * CHUNK, CHUNK)
        for k in range(CHUNK // 16):
            sidx2d[b, pl.ds(k * 16, 16)] = clocal[pl.ds(off + k * 16, 16)]

    def _compute(r, xjv, xiv):
        row16 = iota16 * 16

        def _grp(g, _):
            for e4 in range(4):
                edges = [g * 16 + e4 * 4 + u for u in range(4)]
                ps = [zero16] * 4
                for k in range(C // 16):
                    sl = pl.ds(k * 16, 16)
                    zs = [xjv[ed, sl] + xiv[ed, sl] for ed in edges]
                    lrs = [jnp.maximum(z, NEG * z) for z in zs]
                    ps = [p + att_regs[k] * lr for p, lr in zip(ps, lrs)]
                for u in range(4):
                    ptmp[pl.ds((e4 * 4 + u) * 16, 16)] = ps[u]
            cols = [plsc.load_gather(ptmp, [row16 + cc]) for cc in range(16)]
            while len(cols) > 1:
                cols = [cols[i] + cols[i + 1] for i in range(0, len(cols), 2)]
            t16 = jnp.exp(cols[0])
            off = pl.multiple_of(r * CHUNK, CHUNK)
            lidx = clocal[pl.ds(off + g * 16, 16)]
            plsc.addupdate_scatter(den_l, [lidx], t16, mask=lidx >= 0)
            tvecs = [jnp.broadcast_to(t16[e], (16,)) for e in range(16)]
            for e4 in range(4):
                for k in range(C // 16):
                    sl = pl.ds(k * 16, 16)
                    for u in range(4):
                        edge = g * 16 + e4 * 4 + u
                        xjv[edge, sl] = tvecs[e4 * 4 + u] * xjv[edge, sl]
            return 0

        lax.fori_loop(0, CHUNK // 16, _grp, 0)

    neg16 = jnp.full((16,), -1, jnp.int32)

    def _super(s, _):
        row0 = tile_row0 + s * SUPER
        pltpu.sync_copy(src_hbm.at[pl.ds(row0, SUPER)], src_big)
        pltpu.sync_copy(dst_hbm.at[pl.ds(row0, SUPER)], dst_big)

        def _prefill(i, _):
            sl = pl.ds(i * 16, 16)
            csrc[sl] = neg16
            cdst[sl] = neg16
            clocal[sl] = neg16
            return 0

        lax.fori_loop(0, CMAX // 16, _prefill, 0)

        def _derive(r, cur):
            for g in range(CHUNK // 16):
                sl = pl.ds(g * 16, 16)
                ss = src_big[r, sl]
                dd = dst_big[r, sl]
                local = dd - row_lo
                owned = (local >= 0) & (local < HALF)
                plsc.store_compressed(csrc.at[pl.ds(cur, 16)], ss, mask=owned)
                plsc.store_compressed(cdst.at[pl.ds(cur, 16)], dd, mask=owned)
                plsc.store_compressed(clocal.at[pl.ds(cur, 16)], local,
                                      mask=owned)
                cur = cur + plsc.all_reduce_population_count(owned)[0]
            return cur

        cnt = lax.fori_loop(0, SUPER, _derive, jnp.int32(0))
        npair_d = jnp.maximum(1, (cnt + 2 * CHUNK - 1) // (2 * CHUNK))

        _fire_gather(0, xj_a, xi_a, semj_a, semi_a)
        _fire_gather(1, xj_b, xi_b, semj_b, semi_b)

        def _pair(j, _):
            ra = 2 * j
            rb = 2 * j + 1
            _wait_gather(ra, xj_a, xi_a, semj_a, semi_a)
            _compute(ra, xj_a, xi_a)
            _stage_scatter_idx(ra, 0)
            _scatter_desc(0, xj_a, sems_a).start(add=True)

            _wait_gather(rb, xj_b, xi_b, semj_b, semi_b)
            _compute(rb, xj_b, xi_b)
            _stage_scatter_idx(rb, 1)
            _scatter_desc(1, xj_b, sems_b).start(add=True)

            @pl.when(j < npair_d - 1)
            def _():
                _scatter_desc(0, xj_a, sems_a).wait()
                _fire_gather(ra + 2, xj_a, xi_a, semj_a, semi_a)
                _scatter_desc(1, xj_b, sems_b).wait()
                _fire_gather(rb + 2, xj_b, xi_b, semj_b, semi_b)

            return 0

        lax.fori_loop(0, npair_d, _pair, 0)
        _scatter_desc(0, xj_a, sems_a).wait()
        _scatter_desc(1, xj_b, sems_b).wait()
        return 0

    lax.fori_loop(0, NSUPER, _super, 0)
    plsc.subcore_barrier()

    pltpu.sync_copy(den_l, den_out.at[wid])
    for lo, nrows in acc_slices:
        pltpu.sync_copy(acc_s.at[pl.ds(lo, nrows)], xj_a.at[pl.ds(0, nrows)])
        pltpu.sync_copy(xj_a.at[pl.ds(0, nrows)],
                        acc_out.at[cid, pl.ds(lo, nrows)])


_edge_call = pl.kernel(
    _edge_body,
    out_type=(jax.ShapeDtypeStruct((2, HALF, ACC_C), jnp.float32),
              jax.ShapeDtypeStruct((NW, HALF), jnp.float32)),
    mesh=plsc.VectorSubcoreMesh(core_axis_name="c", subcore_axis_name="s"),
    compiler_params=pltpu.CompilerParams(needs_layout_passes=False),
    scratch_types=[
        pltpu.VMEM((SUPER, CHUNK), jnp.int32),
        pltpu.VMEM((SUPER, CHUNK), jnp.int32),
        pltpu.VMEM((CMAX,), jnp.int32),
        pltpu.VMEM((CMAX,), jnp.int32),
        pltpu.VMEM((CMAX,), jnp.int32),
        pltpu.VMEM((2, CHUNK), jnp.int32),
        pltpu.VMEM((CHUNK, C), jnp.float32),
        pltpu.VMEM((CHUNK, C), jnp.float32),
        pltpu.VMEM((CHUNK, C), jnp.float32),
        pltpu.VMEM((CHUNK, C), jnp.float32),
        pltpu.VMEM((C,), jnp.float32),
        pltpu.VMEM((256,), jnp.float32),
        pltpu.VMEM((HALF,), jnp.float32),
        pltpu.VMEM_SHARED((HALF, ACC_C), jnp.float32),
        pltpu.SemaphoreType.DMA,
        pltpu.SemaphoreType.DMA,
        pltpu.SemaphoreType.DMA,
        pltpu.SemaphoreType.DMA,
        pltpu.SemaphoreType.DMA,
        pltpu.SemaphoreType.DMA,
    ],
)


def _combine_body(acc_ref, den_ref, out_ref):
    d = jnp.sum(den_ref[0], axis=1)
    out_ref[...] = acc_ref[...] / d[:, None]


def _combine(acc_flat, den_t):
    blk = 512
    nblk_half = HALF // blk
    return pl.pallas_call(
        _combine_body,
        grid=(NPAD // blk,),
        in_specs=[
            pl.BlockSpec((blk, ACC_C), lambda i: (i, 0)),
            pl.BlockSpec((1, blk, 16), lambda i: (i // nblk_half,
                                                  i % nblk_half, 0)),
        ],
        out_specs=pl.BlockSpec((blk, C), lambda i: (i, 0)),
        out_shape=jax.ShapeDtypeStruct((NPAD, C), jnp.float32),
    )(acc_flat, den_t)


@jax.jit
def _run(x, edge_index, W_l, W_r, att):
    x_pad = jnp.zeros((NPAD, D), jnp.float32).at[:N].set(x)
    loop = jnp.arange(N, dtype=jnp.int32)
    padidx = N + (jnp.arange(NPADE, dtype=jnp.int32) % 128)
    src = jnp.concatenate([edge_index[0], loop, padidx]).reshape(NROW, CHUNK)
    dst = jnp.concatenate([edge_index[1], loop, padidx]).reshape(NROW, CHUNK)
    xl, xr = _project(x_pad, W_l, W_r)
    acc, den = _edge_call(xl, xr, src, dst, att.reshape(C))
    den_t = den.reshape(2, 16, HALF).transpose(0, 2, 1)
    out_pad = _combine(acc.reshape(NPAD, ACC_C), den_t)
    return out_pad[:N]


def kernel(x, edge_index, W_l, W_r, att):
    return _run(x, edge_index, W_l, W_r, att)


# speedup vs baseline: 1.6521x; 1.6521x over previous
import jax
import jax.numpy as jnp
from jax import lax
from jax.experimental import pallas as pl
from jax.experimental.pallas import tpu as pltpu
from jax.experimental.pallas import tpu_sc as plsc

N = 10000
D = 128
C = 128
E = 320000
NEG = 0.2

NPAD = 10240
HALF = NPAD // 2
ACC_C = 128
NW = 32
CHUNK = 128
Q_CHUNKS = 168
EPAD = 16 * Q_CHUNKS * CHUNK
ETOT = E + N
NPADE = EPAD - ETOT

NROW = EPAD // CHUNK
SUPER = 24
NSUPER = Q_CHUNKS // SUPER
ECHUNK = 64
NBUF = 4


def _proj_body(x_ref, wl_ref, wr_ref, xl_ref, xr_ref):
    xv = x_ref[...]
    xl_ref[...] = jnp.dot(xv, wl_ref[...], preferred_element_type=jnp.float32)
    xr_ref[...] = jnp.dot(xv, wr_ref[...], preferred_element_type=jnp.float32)


def _project(x_pad, W_l, W_r):
    blk = 1280
    return pl.pallas_call(
        _proj_body,
        grid=(NPAD // blk,),
        in_specs=[
            pl.BlockSpec((blk, D), lambda i: (i, 0)),
            pl.BlockSpec((D, C), lambda i: (0, 0)),
            pl.BlockSpec((D, C), lambda i: (0, 0)),
        ],
        out_specs=[
            pl.BlockSpec((blk, C), lambda i: (i, 0)),
            pl.BlockSpec((blk, C), lambda i: (i, 0)),
        ],
        out_shape=[
            jax.ShapeDtypeStruct((NPAD, C), jnp.float32),
            jax.ShapeDtypeStruct((NPAD, C), jnp.float32),
        ],
    )(x_pad, W_l, W_r)


CMAX = SUPER * CHUNK + 4 * ECHUNK


def _edge_body(xl_hbm, xr_hbm, src_hbm, dst_hbm, att_hbm, acc_out, den_out,
               src_big, dst_big, csrc, cdst, clocal, sidx2d,
               xj_0, xi_0, xj_1, xi_1, xj_2, xi_2, xj_3, xi_3,
               att_v, ptmp, den_l, acc_s,
               semj_0, semi_0, sems_0, semj_1, semi_1, sems_1,
               semj_2, semi_2, sems_2, semj_3, semi_3, sems_3):
    cid = lax.axis_index("c")
    sid = lax.axis_index("s")
    wid = cid * 16 + sid
    row_lo = cid * HALF
    tile_row0 = sid * Q_CHUNKS
    zero16 = jnp.zeros((16,), jnp.float32)
    iota16 = lax.iota(jnp.int32, 16)
    acc_slices = [(sid * (HALF // 16), 128),
                  (sid * (HALF // 16) + 128, 128),
                  (sid * (HALF // 16) + 256, 64)]

    bufs = [(xj_0, xi_0, semj_0, semi_0, sems_0),
            (xj_1, xi_1, semj_1, semi_1, sems_1),
            (xj_2, xi_2, semj_2, semi_2, sems_2),
            (xj_3, xi_3, semj_3, semi_3, sems_3)]

    def _zrow(i, _):
        for k in range(ACC_C // 16):
            xj_0[i, pl.ds(k * 16, 16)] = zero16
            xj_1[i, pl.ds(k * 16, 16)] = zero16
        return 0

    lax.fori_loop(0, ECHUNK, _zrow, 0)

    def _zden(i, _):
        den_l[pl.ds(i * 16, 16)] = zero16
        return 0

    lax.fori_loop(0, HALF // 16, _zden, 0)
    for lo, nrows in acc_slices:
        h = nrows // 2
        pltpu.sync_copy(xj_0.at[pl.ds(0, h)], acc_s.at[pl.ds(lo, h)])
        pltpu.sync_copy(xj_1.at[pl.ds(0, h)], acc_s.at[pl.ds(lo + h, h)])
    pltpu.sync_copy(att_hbm, att_v)
    plsc.subcore_barrier()

    att_regs = [att_v[pl.ds(k * 16, 16)] for k in range(C // 16)]

    def _gather_descs(r, xjv, xiv, semj, semi):
        off = pl.multiple_of(r * ECHUNK, ECHUNK)
        dj = pltpu.make_async_copy(
            xl_hbm.at[plsc.Indices(csrc.at[pl.ds(off, ECHUNK)],
                                   ignored_value=-1)],
            xjv, semj)
        di = pltpu.make_async_copy(
            xr_hbm.at[plsc.Indices(cdst.at[pl.ds(off, ECHUNK)],
                                   ignored_value=-1)],
            xiv, semi)
        return dj, di

    def _fire_gather(r, xjv, xiv, semj, semi):
        dj, di = _gather_descs(r, xjv, xiv, semj, semi)
        dj.start()
        di.start()

    def _wait_gather(r, xjv, xiv, semj, semi):
        dj, di = _gather_descs(r, xjv, xiv, semj, semi)
        dj.wait()
        di.wait()

    def _scatter_desc(b, xjv, sems):
        return pltpu.make_async_copy(
            xjv, acc_s.at[plsc.Indices(sidx2d.at[b], ignored_value=-1)],
            sems)

    def _stage_scatter_idx(r, b):
        off = pl.multiple_of(r * ECHUNK, ECHUNK)
        for k in range(ECHUNK // 16):
            sidx2d[b, pl.ds(k * 16, 16)] = clocal[pl.ds(off + k * 16, 16)]

    def _compute(r, xjv, xiv):
        row16 = iota16 * 16

        def _grp(g, _):
            edges = [g * 16 + u for u in range(16)]
            ps = [zero16] * 16
            for k in range(C // 16):
                sl = pl.ds(k * 16, 16)
                zs = [xjv[ed, sl] + xiv[ed, sl] for ed in edges]
                lrs = [jnp.maximum(z, NEG * z) for z in zs]
                ps = [p + att_regs[k] * lr for p, lr in zip(ps, lrs)]
            for u in range(16):
                ptmp[pl.ds(u * 16, 16)] = ps[u]
            cols = [plsc.load_gather(ptmp, [row16 + cc]) for cc in range(16)]
            while len(cols) > 1:
                cols = [cols[i] + cols[i + 1] for i in range(0, len(cols), 2)]
            t16 = jnp.exp(cols[0])
            off = pl.multiple_of(r * ECHUNK, ECHUNK)
            lidx = clocal[pl.ds(off + g * 16, 16)]
            plsc.addupdate_scatter(den_l, [lidx], t16, mask=lidx >= 0)
            tvecs = [jnp.broadcast_to(t16[e], (16,)) for e in range(16)]
            for e4 in range(4):
                for k in range(C // 16):
                    sl = pl.ds(k * 16, 16)
                    for u in range(4):
                        edge = g * 16 + e4 * 4 + u
                        xjv[edge, sl] = tvecs[e4 * 4 + u] * xjv[edge, sl]
            return 0

        lax.fori_loop(0, ECHUNK // 16, _grp, 0)

    neg16 = jnp.full((16,), -1, jnp.int32)

    def _super(s, _):
        row0 = tile_row0 + s * SUPER
        pltpu.sync_copy(src_hbm.at[pl.ds(row0, SUPER)], src_big)
        pltpu.sync_copy(dst_hbm.at[pl.ds(row0, SUPER)], dst_big)

        def _prefill(i, _):
            sl = pl.ds(i * 16, 16)
            csrc[sl] = neg16
            cdst[sl] = neg16
            clocal[sl] = neg16
            return 0

        lax.fori_loop(0, CMAX // 16, _prefill, 0)

        def _derive(r, cur):
            for g in range(CHUNK // 16):
                sl = pl.ds(g * 16, 16)
                ss = src_big[r, sl]
                dd = dst_big[r, sl]
                local = dd - row_lo
                owned = (local >= 0) & (local < HALF)
                plsc.store_compressed(csrc.at[pl.ds(cur, 16)], ss, mask=owned)
                plsc.store_compressed(cdst.at[pl.ds(cur, 16)], dd, mask=owned)
                plsc.store_compressed(clocal.at[pl.ds(cur, 16)], local,
                                      mask=owned)
                cur = cur + plsc.all_reduce_population_count(owned)[0]
            return cur

        cnt = lax.fori_loop(0, SUPER, _derive, jnp.int32(0))
        ntrip = jnp.maximum(1, (cnt + NBUF * ECHUNK - 1) // (NBUF * ECHUNK))

        _fire_gather(0, xj_0, xi_0, semj_0, semi_0)
        _fire_gather(1, xj_1, xi_1, semj_1, semi_1)

        def _trip(t, _):
            for u in range(NBUF):
                xjv, xiv, semj, semi, sems = bufs[u]
                r = NBUF * t + u
                _wait_gather(r, xjv, xiv, semj, semi)
                nxt = bufs[(u + 2) % NBUF]
                if u < 2:
                    @pl.when(t > 0)
                    def _():
                        _scatter_desc((u + 2) % NBUF, nxt[0], nxt[4]).wait()
                    _fire_gather(r + 2, nxt[0], nxt[1], nxt[2], nxt[3])
                else:
                    _scatter_desc((u + 2) % NBUF, nxt[0], nxt[4]).wait()

                    @pl.when(t < ntrip - 1)
                    def _():
                        _fire_gather(r + 2, nxt[0], nxt[1], nxt[2], nxt[3])
                _compute(r, xjv, xiv)
                _stage_scatter_idx(r, u)
                _scatter_desc(u, xjv, sems).start(add=True)
            return 0

        lax.fori_loop(0, ntrip, _trip, 0)
        _scatter_desc(2, xj_2, sems_2).wait()
        _scatter_desc(3, xj_3, sems_3).wait()
        return 0

    lax.fori_loop(0, NSUPER, _super, 0)
    plsc.subcore_barrier()

    pltpu.sync_copy(den_l, den_out.at[wid])
    for lo, nrows in acc_slices:
        h = nrows // 2
        pltpu.sync_copy(acc_s.at[pl.ds(lo, h)], xj_0.at[pl.ds(0, h)])
        pltpu.sync_copy(xj_0.at[pl.ds(0, h)], acc_out.at[cid, pl.ds(lo, h)])
        pltpu.sync_copy(acc_s.at[pl.ds(lo + h, h)], xj_1.at[pl.ds(0, h)])
        pltpu.sync_copy(xj_1.at[pl.ds(0, h)],
                        acc_out.at[cid, pl.ds(lo + h, h)])


_edge_call = pl.kernel(
    _edge_body,
    out_type=(jax.ShapeDtypeStruct((2, HALF, ACC_C), jnp.float32),
              jax.ShapeDtypeStruct((NW, HALF), jnp.float32)),
    mesh=plsc.VectorSubcoreMesh(core_axis_name="c", subcore_axis_name="s"),
    compiler_params=pltpu.CompilerParams(needs_layout_passes=False),
    scratch_types=[
        pltpu.VMEM((SUPER, CHUNK), jnp.int32),
        pltpu.VMEM((SUPER, CHUNK), jnp.int32),
        pltpu.VMEM((CMAX,), jnp.int32),
        pltpu.VMEM((CMAX,), jnp.int32),
        pltpu.VMEM((CMAX,), jnp.int32),
        pltpu.VMEM((NBUF, ECHUNK), jnp.int32),
        pltpu.VMEM((ECHUNK, C), jnp.float32),
        pltpu.VMEM((ECHUNK, C), jnp.float32),
        pltpu.VMEM((ECHUNK, C), jnp.float32),
        pltpu.VMEM((ECHUNK, C), jnp.float32),
        pltpu.VMEM((ECHUNK, C), jnp.float32),
        pltpu.VMEM((ECHUNK, C), jnp.float32),
        pltpu.VMEM((ECHUNK, C), jnp.float32),
        pltpu.VMEM((ECHUNK, C), jnp.float32),
        pltpu.VMEM((C,), jnp.float32),
        pltpu.VMEM((256,), jnp.float32),
        pltpu.VMEM((HALF,), jnp.float32),
        pltpu.VMEM_SHARED((HALF, ACC_C), jnp.float32),
    ] + [pltpu.SemaphoreType.DMA] * 12,
)


def _combine_body(acc_ref, den_ref, out_ref):
    d = jnp.sum(den_ref[0], axis=1)
    out_ref[...] = acc_ref[...] / d[:, None]


def _combine(acc_flat, den_t):
    blk = 512
    nblk_half = HALF // blk
    return pl.pallas_call(
        _combine_body,
        grid=(NPAD // blk,),
        in_specs=[
            pl.BlockSpec((blk, ACC_C), lambda i: (i, 0)),
            pl.BlockSpec((1, blk, 16), lambda i: (i // nblk_half,
                                                  i % nblk_half, 0)),
        ],
        out_specs=pl.BlockSpec((blk, C), lambda i: (i, 0)),
        out_shape=jax.ShapeDtypeStruct((NPAD, C), jnp.float32),
    )(acc_flat, den_t)


@jax.jit
def _run(x, edge_index, W_l, W_r, att):
    x_pad = jnp.zeros((NPAD, D), jnp.float32).at[:N].set(x)
    loop = jnp.arange(N, dtype=jnp.int32)
    pad_src = jnp.zeros((NPADE,), jnp.int32)
    pad_dst = jnp.full((NPADE,), -1, jnp.int32)
    src = jnp.concatenate([edge_index[0], loop, pad_src]).reshape(NROW, CHUNK)
    dst = jnp.concatenate([edge_index[1], loop, pad_dst]).reshape(NROW, CHUNK)
    xl, xr = _project(x_pad, W_l, W_r)
    acc, den = _edge_call(xl, xr, src, dst, att.reshape(C))
    den_t = den.reshape(2, 16, HALF).transpose(0, 2, 1)
    out_pad = _combine(acc.reshape(NPAD, ACC_C), den_t)
    return out_pad[:N]


def kernel(x, edge_index, W_l, W_r, att):
    return _run(x, edge_index, W_l, W_r, att)
